# baseline (speedup 1.0000x reference)
import jax
import jax.numpy as jnp
from jax.experimental import pallas as pl

B = 8
P = 19248
C = 81
M = 32
BP = 3208
NBLK = P // BP
NEG_POS_RATIO = 3
BBOX_W = 1.0
MASK_W = 0.2 / 32.0


def _pass1_body(loc_ref, loct_ref, conf_ref, conft_ref, mask_ref, maskt_ref,
                rank_ref, ce_ref, sl_ref, bce_ref):
    conf = conf_ref[0]
    labels = conft_ref[0]
    amax = jnp.max(conf, axis=-1, keepdims=True)
    ex = jnp.exp(conf - amax)
    lse = jnp.log(jnp.sum(ex, axis=-1, keepdims=True)) + amax
    iota = jax.lax.broadcasted_iota(jnp.int32, (BP, C), 1)
    xl = jnp.sum(jnp.where(iota == labels, conf, 0.0), axis=-1, keepdims=True)
    rank_ref[0] = lse - conf[:, 0:1]
    ce_ref[0] = lse - xl

    d = jnp.abs(loc_ref[0] - loct_ref[0])
    sl1 = jnp.where(d < 1.0, 0.5 * d * d, d - 0.5)
    sl_ref[0] = jnp.sum(sl1, axis=-1, keepdims=True)

    p = jnp.clip(mask_ref[0], 1e-7, 1.0 - 1e-7)
    mt = maskt_ref[0]
    a = jnp.log(p)
    b = jnp.log1p(-p)
    bce = mt * (b - a) - b
    bce_ref[0] = jnp.sum(bce, axis=-1, keepdims=True)


def _select_body(rank_ref, ce_ref, sl_ref, bce_ref, conft_ref, out_ref):
    labels = conft_ref[...]
    pos = labels > 0
    skip = labels <= 0
    posf = jnp.where(pos, 1.0, 0.0)
    inv = labels < 0

    ce = ce_ref[...]
    s = jnp.where(pos | inv, 0.0, rank_ref[...])
    cenv = jnp.where(pos | inv, 0.0, ce)

    np_rows = jnp.sum(posf, axis=1, keepdims=True)
    np_total = jnp.sum(np_rows)
    l_loc_tot = jnp.sum(sl_ref[...] * posf)
    l_mask_tot = jnp.sum(bce_ref[...] * posf)
    ce_pos_tot = jnp.sum(ce * posf)

    k = jnp.minimum(NEG_POS_RATIO * np_rows.astype(jnp.int32), P - 1)
    bits = jax.lax.bitcast_convert_type(s, jnp.int32)

    def t_step(i, pref):
        cand = pref | (jnp.int32(1) << (30 - i))
        cnt = jnp.sum((bits >= cand).astype(jnp.int32), axis=1, keepdims=True)
        return jnp.where(cnt >= k, cand, pref)

    t = jax.lax.fori_loop(0, 31, t_step, jnp.zeros((B, 1), jnp.int32))

    cgt = jnp.sum((bits > t).astype(jnp.int32), axis=1, keepdims=True)
    rem = k - cgt
    tie = bits == t
    idx = jax.lax.broadcasted_iota(jnp.int32, (B, P), 1)

    def j_step(i, acc):
        cand = acc | (jnp.int32(1) << (14 - i))
        cnt = jnp.sum((tie & (idx < cand)).astype(jnp.int32),
                      axis=1, keepdims=True)
        return jnp.where(cnt <= rem, cand, acc)

    j_lim = jax.lax.fori_loop(0, 15, j_step, jnp.zeros((B, 1), jnp.int32))

    sel = (bits > t) | (tie & (idx < j_lim))
    neg_sum = jnp.sum(jnp.where(sel, cenv, 0.0))

    n = jnp.maximum(np_total, 1.0)
    loss_l = l_loc_tot * BBOX_W / n
    loss_c = (ce_pos_tot + neg_sum) / n
    loss_m = l_mask_tot * MASK_W / n
    ones = jnp.ones((1, 128), jnp.float32)
    out_ref[...] = jnp.concatenate(
        [loss_l * ones, loss_c * ones, loss_m * ones,
         jnp.zeros((5, 128), jnp.float32)], axis=0)


def _run(loc_data, conf_data, mask_data, loc_t, conf_t, masks_t,
         interpret=False):
    conf_t3 = conf_t.reshape(B, P, 1)
    pp = pl.BlockSpec((1, BP, 1), lambda b, j: (b, j, 0))
    rank, ce, sl, bce = pl.pallas_call(
        _pass1_body,
        grid=(B, NBLK),
        in_specs=[
            pl.BlockSpec((1, BP, 4), lambda b, j: (b, j, 0)),
            pl.BlockSpec((1, BP, 4), lambda b, j: (b, j, 0)),
            pl.BlockSpec((1, BP, C), lambda b, j: (b, j, 0)),
            pl.BlockSpec((1, BP, 1), lambda b, j: (b, j, 0)),
            pl.BlockSpec((1, BP, M), lambda b, j: (b, j, 0)),
            pl.BlockSpec((1, BP, M), lambda b, j: (b, j, 0)),
        ],
        out_specs=[pp, pp, pp, pp],
        out_shape=[jax.ShapeDtypeStruct((B, P, 1), jnp.float32)] * 4,
        interpret=interpret,
    )(loc_data, loc_t, conf_data, conf_t3, mask_data, masks_t)

    out = pl.pallas_call(
        _select_body,
        out_shape=jax.ShapeDtypeStruct((8, 128), jnp.float32),
        interpret=interpret,
    )(rank.reshape(B, P), ce.reshape(B, P), sl.reshape(B, P),
      bce.reshape(B, P), conf_t)
    return (out[0, 0], out[1, 0], out[2, 0])


def kernel(loc_data, conf_data, mask_data, loc_t, conf_t, masks_t):
    return _run(loc_data, conf_data, mask_data, loc_t, conf_t, masks_t)
